# baseline (speedup 1.0000x reference)
import functools

import jax
import jax.numpy as jnp
from jax import lax
from jax.experimental import pallas as pl
from jax.experimental.pallas import tpu as pltpu
from jax.experimental.pallas import tpu_sc as plsc

PAD = -2
SIGN = -(2**31)
MASK31 = 2**31 - 1


@functools.lru_cache(maxsize=None)
def _build(B, L, V, D):
    N = B * L
    info = plsc.get_sparse_core_info()
    NC, NS, LANES = info.num_cores, info.num_subcores, info.num_lanes
    NW = NC * NS
    assert N % NW == 0
    per_w = N // NW
    b_per_w = B // NW
    GCH = 128
    BB = 2
    BLK = BB * L
    assert BLK % LANES == 0 and per_w % BLK == 0
    nb = per_w // BLK
    S = 5
    RING = S * GCH

    mesh = plsc.VectorSubcoreMesh(core_axis_name="c", subcore_axis_name="s")

    @functools.partial(
        pl.kernel,
        out_type=jax.ShapeDtypeStruct((B, L, D), jnp.float32),
        mesh=mesh,
        compiler_params=pltpu.CompilerParams(
            needs_layout_passes=False, use_tc_tiling_on_sc=False
        ),
        scratch_types=[
            pltpu.VMEM((per_w,), jnp.int32),
            pltpu.VMEM((per_w,), jnp.int32),
            pltpu.VMEM((RING, D), jnp.float32),
            pltpu.VMEM((BB, L, D), jnp.float32),
            pltpu.SMEM((nb + 1,), jnp.int32),
            pltpu.SemaphoreType.DMA,
        ],
    )
    def k(tok_hbm, table_hbm, out_hbm, meta_v, cidx_v, stage_v, oblk_v,
          pbnd_s, sem_g):
        wid = lax.axis_index("s") * NC + lax.axis_index("c")
        base = wid * per_w
        bbase = wid * b_per_w
        lane = lax.iota(jnp.int32, LANES)
        zeros = jnp.zeros((LANES,), jnp.float32)
        izeros = jnp.zeros((LANES,), jnp.int32)
        cols = [jnp.full((LANES,), c, jnp.int32) for c in range(D)]

        pltpu.sync_copy(tok_hbm.at[pl.ds(base, per_w)], meta_v)

        def zero_cidx(i, _):
            cidx_v[pl.ds(i * LANES, LANES)] = izeros
            return 0

        lax.fori_loop(0, per_w // LANES, zero_cidx, 0)

        def prep(t, running):
            pbnd_s[t] = running
            for u in range(BLK // LANES):
                off = t * BLK + u * LANES
                v = meta_v[pl.ds(off, LANES)]
                m = v >= 0
                mi = m.astype(jnp.int32)
                cs = plsc.cumsum(mi)
                pexcl = running + cs - mi
                plsc.store_compressed(
                    cidx_v.at[pl.ds(running, LANES)],
                    jnp.maximum(v, 0),
                    mask=m,
                )
                meta_v[pl.ds(off, LANES)] = pexcl | jnp.where(m, 0, SIGN)
                running = running + cs[15]
            return running

        nv = lax.fori_loop(0, nb, prep, jnp.int32(0))
        pbnd_s[nb] = nv
        nch = (nv + GCH - 1) // GCH

        def g_desc(c):
            s = lax.rem(c, S)
            return pltpu.make_async_copy(
                table_hbm.at[cidx_v.at[pl.ds(c * GCH, GCH)]],
                stage_v.at[pl.ds(s * GCH, GCH)],
                sem_g,
            )

        def block(t, carry):
            fired, waited = carry
            limit = jnp.minimum(nch, pbnd_s[t] // GCH + S)

            def fire_body(f):
                g_desc(f).start()
                return f + 1

            fired = lax.while_loop(lambda f: f < limit, fire_body, fired)

            need = (pbnd_s[t + 1] + GCH - 1) // GCH

            def wait_body(w):
                g_desc(w).wait()
                return w + 1

            waited = lax.while_loop(lambda w: w < need, wait_body, waited)

            for gg in range(BLK // LANES):
                enc = meta_v[pl.ds(t * BLK + gg * LANES, LANES)]
                m = enc >= 0
                rowids = gg * LANES + lane
                b2 = rowids // L
                lv = rowids - b2 * L
                nval = plsc.all_reduce_population_count(m)[0]

                @pl.when(nval > 0)
                def _():
                    ring = lax.rem(enc & MASK31, RING)
                    for c in range(D):
                        vals = plsc.load_gather(stage_v, [ring, cols[c]])
                        plsc.store_scatter(
                            oblk_v, [b2, lv, cols[c]], vals, mask=m
                        )

                @pl.when(nval < LANES)
                def _():
                    mz = jnp.logical_not(m)
                    for c in range(D):
                        plsc.store_scatter(
                            oblk_v, [b2, lv, cols[c]], zeros, mask=mz
                        )

            pltpu.sync_copy(oblk_v, out_hbm.at[pl.ds(bbase + t * BB, BB)])
            return fired, waited

        lax.fori_loop(0, nb, block, (jnp.int32(0), jnp.int32(0)))

    return k


def kernel(ecg_tokens, emb_table):
    B, L = ecg_tokens.shape
    V, D = emb_table.shape
    k = _build(B, L, V, D)
    return k(ecg_tokens.reshape(B * L), emb_table)


# speedup vs baseline: 1.1770x; 1.1770x over previous
import functools

import jax
import jax.numpy as jnp
from jax import lax
from jax.experimental import pallas as pl
from jax.experimental.pallas import tpu as pltpu
from jax.experimental.pallas import tpu_sc as plsc

PAD = -2
SIGN = -(2**31)
MASK31 = 2**31 - 1


@functools.lru_cache(maxsize=None)
def _build(N, V, D):
    info = plsc.get_sparse_core_info()
    NC, NS, LANES = info.num_cores, info.num_subcores, info.num_lanes
    NW = NC * NS
    assert N % NW == 0
    per_w = N // NW
    GCH = 128
    assert per_w % GCH == 0
    nb = per_w // GCH
    S = 4
    RING = S * GCH

    mesh = plsc.VectorSubcoreMesh(core_axis_name="c", subcore_axis_name="s")

    @functools.partial(
        pl.kernel,
        out_type=jax.ShapeDtypeStruct((N, D), jnp.float32),
        mesh=mesh,
        compiler_params=pltpu.CompilerParams(
            needs_layout_passes=False, use_tc_tiling_on_sc=False
        ),
        scratch_types=[
            pltpu.VMEM((per_w,), jnp.int32),
            pltpu.VMEM((per_w,), jnp.int32),
            pltpu.VMEM((RING, D), jnp.float32),
            pltpu.VMEM((2 * GCH, D), jnp.float32),
            pltpu.SMEM((nb + 1,), jnp.int32),
            pltpu.SemaphoreType.DMA,
            pltpu.SemaphoreType.DMA,
        ],
    )
    def k(tok_hbm, table_hbm, out_hbm, meta_v, cidx_v, stage_v, oblk_v,
          pbnd_s, sem_g, sem_o):
        wid = lax.axis_index("s") * NC + lax.axis_index("c")
        base = wid * per_w
        lane = lax.iota(jnp.int32, LANES)
        zeros = jnp.zeros((LANES,), jnp.float32)
        izeros = jnp.zeros((LANES,), jnp.int32)
        cols = [jnp.full((LANES,), c, jnp.int32) for c in range(D)]

        pltpu.sync_copy(tok_hbm.at[pl.ds(base, per_w)], meta_v)

        def zero_cidx(i, _):
            cidx_v[pl.ds(i * LANES, LANES)] = izeros
            return 0

        lax.fori_loop(0, per_w // LANES, zero_cidx, 0)

        def prep(t, running):
            pbnd_s[t] = running
            for u in range(GCH // LANES):
                off = t * GCH + u * LANES
                v = meta_v[pl.ds(off, LANES)]
                m = v >= 0
                mi = m.astype(jnp.int32)
                cs = plsc.cumsum(mi)
                pexcl = running + cs - mi
                plsc.store_compressed(
                    cidx_v.at[pl.ds(running, LANES)], jnp.maximum(v, 0), mask=m
                )
                meta_v[pl.ds(off, LANES)] = pexcl | jnp.where(m, 0, SIGN)
                running = running + cs[15]
            return running

        nv = lax.fori_loop(0, nb, prep, jnp.int32(0))
        pbnd_s[nb] = nv
        nch = (nv + GCH - 1) // GCH

        def g_desc(c):
            s = lax.rem(c, S)
            return pltpu.make_async_copy(
                table_hbm.at[cidx_v.at[pl.ds(c * GCH, GCH)]],
                stage_v.at[pl.ds(s * GCH, GCH)],
                sem_g,
            )

        def o_desc(t):
            s = lax.rem(t, 2)
            return pltpu.make_async_copy(
                oblk_v.at[pl.ds(s * GCH, GCH)],
                out_hbm.at[pl.ds(base + t * GCH, GCH)],
                sem_o,
            )

        def block(t, carry):
            fired, waited = carry
            limit = jnp.minimum(nch, pbnd_s[t] // GCH + S)

            def fire_body(f):
                g_desc(f).start()
                return f + 1

            fired = lax.while_loop(lambda f: f < limit, fire_body, fired)

            need = (pbnd_s[t + 1] + GCH - 1) // GCH

            def wait_body(w):
                g_desc(w).wait()
                return w + 1

            waited = lax.while_loop(lambda w: w < need, wait_body, waited)

            @pl.when(t >= 2)
            def _():
                o_desc(t - 2).wait()

            obase = lax.rem(t, 2) * GCH

            def zfill(q, _):
                r = obase + q
                for c in range(D // LANES):
                    oblk_v[r, pl.ds(c * LANES, LANES)] = zeros
                return 0

            lax.fori_loop(0, GCH, zfill, 0)

            for gg in range(GCH // LANES):
                enc = meta_v[pl.ds(t * GCH + gg * LANES, LANES)]
                m = enc >= 0
                ring = enc & (RING - 1)
                rowids = obase + gg * LANES + lane
                for c in range(D):
                    vals = plsc.load_gather(stage_v, [ring, cols[c]])
                    plsc.store_scatter(
                        oblk_v, [rowids, cols[c]], vals, mask=m
                    )

            o_desc(t).start()
            return fired, waited

        lax.fori_loop(0, nb, block, (jnp.int32(0), jnp.int32(0)))
        o_desc(nb - 2).wait()
        o_desc(nb - 1).wait()

    return k


def kernel(ecg_tokens, emb_table):
    B, L = ecg_tokens.shape
    V, D = emb_table.shape
    N = B * L
    k = _build(N, V, D)
    out = k(ecg_tokens.reshape(N), emb_table)
    return out.reshape(B, L, D)


# speedup vs baseline: 2.0008x; 1.7000x over previous
import functools

import jax
import jax.numpy as jnp
from jax import lax
from jax.experimental import pallas as pl
from jax.experimental.pallas import tpu as pltpu
from jax.experimental.pallas import tpu_sc as plsc

PAD = -2
SIGN = -(2**31)
MASK31 = 2**31 - 1


@functools.lru_cache(maxsize=None)
def _build(N, V, D):
    info = plsc.get_sparse_core_info()
    NC, NS, LANES = info.num_cores, info.num_subcores, info.num_lanes
    NW = NC * NS
    assert N % NW == 0
    per_w = N // NW
    GCH = 128
    assert per_w % GCH == 0
    nb = per_w // GCH
    S = 4
    RING = S * GCH

    mesh = plsc.VectorSubcoreMesh(core_axis_name="c", subcore_axis_name="s")

    @functools.partial(
        pl.kernel,
        out_type=jax.ShapeDtypeStruct((N, D), jnp.float32),
        mesh=mesh,
        compiler_params=pltpu.CompilerParams(
            needs_layout_passes=False, use_tc_tiling_on_sc=False
        ),
        scratch_types=[
            pltpu.VMEM((per_w,), jnp.int32),
            pltpu.VMEM((per_w,), jnp.int32),
            pltpu.VMEM((per_w + 16,), jnp.int32),
            pltpu.VMEM((RING, D), jnp.float32),
            pltpu.VMEM((2 * GCH, D), jnp.float32),
            pltpu.SMEM((nb + 1,), jnp.int32),
            pltpu.SemaphoreType.DMA,
            pltpu.SemaphoreType.DMA,
        ],
    )
    def k(tok_hbm, table_hbm, out_hbm, meta_v, cidx_v, ridx_v, stage_v,
          oblk_v, pbnd_s, sem_g, sem_o):
        wid = lax.axis_index("s") * NC + lax.axis_index("c")
        base = wid * per_w
        lane = lax.iota(jnp.int32, LANES)
        zeros = jnp.zeros((LANES,), jnp.float32)
        izeros = jnp.zeros((LANES,), jnp.int32)
        cols = [jnp.full((LANES,), c, jnp.int32) for c in range(D)]

        pltpu.sync_copy(tok_hbm.at[pl.ds(base, per_w)], meta_v)

        def zero_cidx(i, _):
            cidx_v[pl.ds(i * LANES, LANES)] = izeros
            return 0

        lax.fori_loop(0, per_w // LANES, zero_cidx, 0)

        def prep(t, running):
            pbnd_s[t] = running
            for u in range(GCH // LANES):
                off = t * GCH + u * LANES
                v = meta_v[pl.ds(off, LANES)]
                m = v >= 0
                plsc.store_compressed(
                    cidx_v.at[pl.ds(running, LANES)], jnp.maximum(v, 0), mask=m
                )
                plsc.store_compressed(
                    ridx_v.at[pl.ds(running, LANES)], off + lane, mask=m
                )
                running = running + plsc.all_reduce_population_count(m)[0]
            return running

        nv = lax.fori_loop(0, nb, prep, jnp.int32(0))
        pbnd_s[nb] = nv
        nch = (nv + GCH - 1) // GCH

        def g_desc(c):
            s = lax.rem(c, S)
            return pltpu.make_async_copy(
                table_hbm.at[cidx_v.at[pl.ds(c * GCH, GCH)]],
                stage_v.at[pl.ds(s * GCH, GCH)],
                sem_g,
            )

        def o_desc(t):
            s = lax.rem(t, 2)
            return pltpu.make_async_copy(
                oblk_v.at[pl.ds(s * GCH, GCH)],
                out_hbm.at[pl.ds(base + t * GCH, GCH)],
                sem_o,
            )

        def block(t, carry):
            fired, waited = carry
            limit = jnp.minimum(nch, pbnd_s[t] // GCH + S)

            def fire_body(f):
                g_desc(f).start()
                return f + 1

            fired = lax.while_loop(lambda f: f < limit, fire_body, fired)

            need = (pbnd_s[t + 1] + GCH - 1) // GCH

            def wait_body(w):
                g_desc(w).wait()
                return w + 1

            waited = lax.while_loop(lambda w: w < need, wait_body, waited)

            @pl.when(t >= 2)
            def _():
                o_desc(t - 2).wait()

            obase = lax.rem(t, 2) * GCH

            def zfill(q, _):
                r = obase + q
                for c in range(D // LANES):
                    oblk_v[r, pl.ds(c * LANES, LANES)] = zeros
                return 0

            lax.fori_loop(0, GCH, zfill, 0)

            pstart = pbnd_s[t]
            pend = pbnd_s[t + 1]
            roff = obase - t * GCH

            def copy_row(rv, p, l):
                r = rv[l] + roff
                sr = (p + l) & (RING - 1)
                for c in range(D // LANES):
                    oblk_v[r, pl.ds(c * LANES, LANES)] = stage_v[
                        sr, pl.ds(c * LANES, LANES)
                    ]

            def full_grp(p):
                rv = ridx_v[pl.ds(p, LANES)]
                for l in range(LANES):
                    copy_row(rv, p, l)
                return p + LANES

            p0 = lax.while_loop(
                lambda p: p + LANES <= pend, full_grp, pstart
            )
            rem_n = pend - p0
            rv_t = ridx_v[pl.ds(p0, LANES)]
            for l in range(LANES):
                @pl.when(l < rem_n)
                def _():
                    copy_row(rv_t, p0, l)

            o_desc(t).start()
            return fired, waited

        lax.fori_loop(0, nb, block, (jnp.int32(0), jnp.int32(0)))
        o_desc(nb - 2).wait()
        o_desc(nb - 1).wait()

    return k


def kernel(ecg_tokens, emb_table):
    B, L = ecg_tokens.shape
    V, D = emb_table.shape
    N = B * L
    k = _build(N, V, D)
    out = k(ecg_tokens.reshape(N), emb_table)
    return out.reshape(B, L, D)


# speedup vs baseline: 2.0010x; 1.0001x over previous
import functools

import jax
import jax.numpy as jnp
from jax import lax
from jax.experimental import pallas as pl
from jax.experimental.pallas import tpu as pltpu
from jax.experimental.pallas import tpu_sc as plsc

PAD = -2


@functools.lru_cache(maxsize=None)
def _build(N, V, D):
    info = plsc.get_sparse_core_info()
    NC, NS, LANES = info.num_cores, info.num_subcores, info.num_lanes
    NW = NC * NS
    assert N % NW == 0
    per_w = N // NW
    GCH = 128
    assert per_w % GCH == 0
    nb = per_w // GCH
    S = 4
    RING = S * GCH

    mesh = plsc.VectorSubcoreMesh(core_axis_name="c", subcore_axis_name="s")

    @functools.partial(
        pl.kernel,
        out_type=jax.ShapeDtypeStruct((N, D), jnp.float32),
        mesh=mesh,
        compiler_params=pltpu.CompilerParams(
            needs_layout_passes=False, use_tc_tiling_on_sc=False
        ),
        scratch_types=[
            pltpu.VMEM((per_w,), jnp.int32),
            pltpu.VMEM((per_w,), jnp.int32),
            pltpu.VMEM((per_w + 16,), jnp.int32),
            pltpu.VMEM((RING, D), jnp.float32),
            pltpu.VMEM((2 * GCH, D), jnp.float32),
            pltpu.SMEM((nb + 1,), jnp.int32),
            pltpu.SemaphoreType.DMA,
            pltpu.SemaphoreType.DMA,
        ],
    )
    def k(tok_hbm, table_hbm, out_hbm, meta_v, cidx_v, ridx_v, stage_v,
          oblk_v, pbnd_s, sem_g, sem_o):
        wid = lax.axis_index("s") * NC + lax.axis_index("c")
        base = wid * per_w
        lane = lax.iota(jnp.int32, LANES)
        zeros = jnp.zeros((LANES,), jnp.float32)
        izeros = jnp.zeros((LANES,), jnp.int32)

        pltpu.sync_copy(tok_hbm.at[pl.ds(base, per_w)], meta_v)

        def zero_cidx(i, _):
            cidx_v[pl.ds(i * LANES, LANES)] = izeros
            return 0

        lax.fori_loop(0, per_w // LANES, zero_cidx, 0)

        def prep(t, running):
            pbnd_s[t] = running
            for u in range(GCH // LANES):
                off = t * GCH + u * LANES
                v = meta_v[pl.ds(off, LANES)]
                m = v >= 0
                plsc.store_compressed(
                    cidx_v.at[pl.ds(running, LANES)], jnp.maximum(v, 0), mask=m
                )
                plsc.store_compressed(
                    ridx_v.at[pl.ds(running, LANES)], off + lane, mask=m
                )
                running = running + plsc.all_reduce_population_count(m)[0]
            return running

        nv = lax.fori_loop(0, nb, prep, jnp.int32(0))
        pbnd_s[nb] = nv
        nch = (nv + GCH - 1) // GCH

        def g_desc(c):
            s = lax.rem(c, S)
            return pltpu.make_async_copy(
                table_hbm.at[cidx_v.at[pl.ds(c * GCH, GCH)]],
                stage_v.at[pl.ds(s * GCH, GCH)],
                sem_g,
            )

        def o_desc(t):
            s = lax.rem(t, 2)
            return pltpu.make_async_copy(
                oblk_v.at[pl.ds(s * GCH, GCH)],
                out_hbm.at[pl.ds(base + t * GCH, GCH)],
                sem_o,
            )

        def block(t, carry):
            fired, waited = carry
            limit = jnp.minimum(nch, pbnd_s[t] // GCH + S)

            def fire_body(f):
                g_desc(f).start()
                return f + 1

            fired = lax.while_loop(lambda f: f < limit, fire_body, fired)

            need = (pbnd_s[t + 1] + GCH - 1) // GCH

            def wait_body(w):
                g_desc(w).wait()
                return w + 1

            waited = lax.while_loop(lambda w: w < need, wait_body, waited)

            @pl.when(t >= 2)
            def _():
                o_desc(t - 2).wait()

            obase = lax.rem(t, 2) * GCH

            def zfill(q, _):
                r = obase + q
                for c in range(D // LANES):
                    oblk_v[r, pl.ds(c * LANES, LANES)] = zeros
                return 0

            lax.fori_loop(0, GCH, zfill, 0)

            pstart = pbnd_s[t]
            pend = pbnd_s[t + 1]
            roff = obase - t * GCH

            def copy_row(rv, p, l):
                r = rv[l] + roff
                sr = (p + l) & (RING - 1)
                for c in range(D // LANES):
                    oblk_v[r, pl.ds(c * LANES, LANES)] = stage_v[
                        sr, pl.ds(c * LANES, LANES)
                    ]

            def full_grp(p):
                rv = ridx_v[pl.ds(p, LANES)]
                for l in range(LANES):
                    copy_row(rv, p, l)
                return p + LANES

            p0 = lax.while_loop(
                lambda p: p + LANES <= pend, full_grp, pstart
            )
            rem_n = pend - p0
            rv_t = ridx_v[pl.ds(p0, LANES)]
            for l in range(LANES):
                @pl.when(l < rem_n)
                def _():
                    copy_row(rv_t, p0, l)

            o_desc(t).start()
            return fired, waited

        lax.fori_loop(0, nb, block, (jnp.int32(0), jnp.int32(0)))
        o_desc(nb - 2).wait()
        o_desc(nb - 1).wait()

    return k


def kernel(ecg_tokens, emb_table):
    B, L = ecg_tokens.shape
    V, D = emb_table.shape
    N = B * L
    k = _build(N, V, D)
    out = k(ecg_tokens.reshape(N), emb_table)
    return out.reshape(B, L, D)
